# Initial kernel scaffold; baseline (speedup 1.0000x reference)
import jax
import jax.numpy as jnp
from jax.experimental import pallas as pl


def kernel(x_, edge_index, edge_weight, z, Wt0, bt0, Wt1, bt1, Wc0, bc0, Wc1, bc1, gn_w, gn_b, gn_ms):
    raise NotImplementedError("write your pallas kernel here")


# speedup vs baseline: 5.5843x; 5.5843x over previous
import dataclasses
import functools

import jax
import jax.numpy as jnp
from jax import lax
from jax.experimental import pallas as pl
from jax.experimental.pallas import tpu as pltpu
from jax.experimental.pallas import tpu_sc as plsc

N = 10000
E = 320000
D = 128
ZR = 0.8
EPS = 1e-5

NC = 2
NS = 16
DH = D // NC
EPT = E // NS
C = 80
NCHUNK = EPT // C
NPAD = 10240
RPT = NPAD // NS
DW = 16


def _pre_body(x_ref, z_ref, wt0_ref, bt0_ref, wt1_ref, bt1_ref,
              xm_ref, xself_ref):
    x = x_ref[...]
    x0 = jnp.maximum(jnp.dot(x, wt0_ref[...],
                             preferred_element_type=jnp.float32) + bt0_ref[...], 0.0)
    x1 = jnp.maximum(jnp.dot(x, wt1_ref[...],
                             preferred_element_type=jnp.float32) + bt1_ref[...], 0.0)
    x_out = ZR * x0 + (1.0 - ZR) * x1
    x_in = ZR * x1 + (1.0 - ZR) * x0
    mask = z_ref[...] > 0.5
    xm_ref[:N, :] = x_out[:, :DH]
    xm_ref[N:, :] = x_out[:, DH:]
    xself_ref[...] = jnp.where(mask, x_in, x_out)


def _post_body(accp_ref, degp_ref, xself_ref, z_ref,
               wc1a_ref, wc1b_ref, bc1_ref, wc0a_ref, wc0b_ref, bc0_ref,
               gnw_ref, gnb_ref, gnms_ref, out_ref):
    acc = jnp.concatenate([accp_ref[:N, :], accp_ref[NPAD:NPAD + N, :]], axis=1)
    deg = degp_ref[:N, :1]
    deg = jnp.where(deg < 0.5, deg + 1.0, deg)
    out = acc / deg
    mean = jnp.mean(out, axis=0, keepdims=True)
    o = out - mean * gnms_ref[...]
    var = jnp.mean(o * o, axis=0, keepdims=True)
    outn = gnw_ref[...] * o * lax.rsqrt(var + EPS) + gnb_ref[...]
    xself = xself_ref[...]
    y1 = (jnp.dot(outn, wc1a_ref[...], preferred_element_type=jnp.float32)
          + jnp.dot(xself, wc1b_ref[...], preferred_element_type=jnp.float32)
          + bc1_ref[...])
    y0 = (jnp.dot(outn, wc0a_ref[...], preferred_element_type=jnp.float32)
          + jnp.dot(xself, wc0b_ref[...], preferred_element_type=jnp.float32)
          + bc0_ref[...])
    y_in = ZR * y1 + (1.0 - ZR) * y0
    y_out = ZR * y0 + (1.0 - ZR) * y1
    mask = z_ref[...] > 0.5
    out_ref[...] = jnp.where(mask, y_in, y_out)


_sc_mesh = plsc.VectorSubcoreMesh(core_axis_name="c", subcore_axis_name="s")

_sc_params = pltpu.CompilerParams(use_tc_tiling_on_sc=False)
if "needs_layout_passes" in pltpu.CompilerParams.__dataclass_fields__:
    _sc_params = dataclasses.replace(_sc_params, needs_layout_passes=False)


@functools.partial(
    pl.kernel,
    compiler_params=_sc_params,
    out_type=(jax.ShapeDtypeStruct((NC * NPAD, DH), jnp.float32),
              jax.ShapeDtypeStruct((NC * NPAD, DW), jnp.float32)),
    mesh=_sc_mesh,
    scratch_types=[
        pltpu.VMEM((C,), jnp.int32),
        pltpu.VMEM((C,), jnp.int32),
        pltpu.VMEM((C,), jnp.float32),
        pltpu.VMEM((C, DH), jnp.float32),
        pltpu.VMEM((C, DW), jnp.float32),
        pltpu.VMEM((RPT, DH), jnp.float32),
        pltpu.VMEM((RPT, DW), jnp.float32),
        pltpu.VMEM_SHARED((NPAD, DH), jnp.float32),
        pltpu.VMEM_SHARED((NPAD, DW), jnp.float32),
    ],
)
def _sc_scatter(xm_hbm, row_hbm, col_hbm, w_hbm, zrow_hbm, zdeg_hbm,
                accp_hbm, degp_hbm,
                colv, rowv, wv, msgv, wrowv, bounce, dbounce, acc_sh, deg_sh):
    cid = lax.axis_index("c")
    sid = lax.axis_index("s")

    pltpu.sync_copy(zrow_hbm, bounce)
    pltpu.sync_copy(zdeg_hbm, dbounce)
    pltpu.sync_copy(bounce, acc_sh.at[pl.ds(sid * RPT, RPT)])
    pltpu.sync_copy(dbounce, deg_sh.at[pl.ds(sid * RPT, RPT)])
    plsc.subcore_barrier()

    base0 = sid * EPT
    half = cid * N

    @pl.loop(0, NCHUNK)
    def _chunk(ci):
        base = base0 + ci * C
        pltpu.sync_copy(col_hbm.at[pl.ds(base, C)], colv)
        pltpu.sync_copy(row_hbm.at[pl.ds(base, C)], rowv)
        pltpu.sync_copy(w_hbm.at[pl.ds(base, C)], wv)

        @pl.loop(0, C, step=16)
        def _shift(k):
            sl = pl.ds(k, 16)
            colv[sl] = colv[sl] + half

        pltpu.sync_copy(xm_hbm.at[colv], msgv)

        @pl.loop(0, C)
        def _edge(e):
            w16 = plsc.load_gather(wv, [jnp.full((16,), e, jnp.int32)])
            wrowv[e, :] = w16
            for d in range(DH // 16):
                sl = pl.ds(d * 16, 16)
                msgv[e, sl] = msgv[e, sl] * w16

        pltpu.sync_copy(msgv, acc_sh.at[rowv], add=True)
        pltpu.sync_copy(wrowv, deg_sh.at[rowv], add=True)

    plsc.subcore_barrier()

    obase = cid * NPAD + sid * RPT
    pltpu.sync_copy(acc_sh.at[pl.ds(sid * RPT, RPT)], bounce)
    pltpu.sync_copy(deg_sh.at[pl.ds(sid * RPT, RPT)], dbounce)
    pltpu.sync_copy(bounce, accp_hbm.at[pl.ds(obase, RPT)])
    pltpu.sync_copy(dbounce, degp_hbm.at[pl.ds(obase, RPT)])


def kernel(x_, edge_index, edge_weight, z, Wt0, bt0, Wt1, bt1,
           Wc0, bc0, Wc1, bc1, gn_w, gn_b, gn_ms):
    row = edge_index[0].astype(jnp.int32)
    col = edge_index[1].astype(jnp.int32)
    z2 = z.reshape(N, 1)

    xm2, xself = pl.pallas_call(
        _pre_body,
        out_shape=(jax.ShapeDtypeStruct((NC * N, DH), jnp.float32),
                   jax.ShapeDtypeStruct((N, D), jnp.float32)),
    )(x_, z2, Wt0, bt0.reshape(1, D), Wt1, bt1.reshape(1, D))

    zrow = jnp.zeros((RPT, DH), jnp.float32)
    zdeg = jnp.zeros((RPT, DW), jnp.float32)
    accp, degp = _sc_scatter(xm2, row, col, edge_weight, zrow, zdeg)

    out = pl.pallas_call(
        _post_body,
        out_shape=jax.ShapeDtypeStruct((N, D), jnp.float32),
    )(accp, degp, xself, z2,
      Wc1[:D], Wc1[D:], bc1.reshape(1, D),
      Wc0[:D], Wc0[D:], bc0.reshape(1, D),
      gn_w.reshape(1, D), gn_b.reshape(1, D), gn_ms.reshape(1, D))
    return out


# speedup vs baseline: 9.9847x; 1.7880x over previous
import dataclasses
import functools

import jax
import jax.numpy as jnp
from jax import lax
from jax.experimental import pallas as pl
from jax.experimental.pallas import tpu as pltpu
from jax.experimental.pallas import tpu_sc as plsc

N = 10000
E = 320000
D = 128
ZR = 0.8
EPS = 1e-5

NC = 2
NS = 16
DH = D // NC
EPT = E // NS
C = 80
NCHUNK = EPT // C
RING = 5
NGROUP = NCHUNK // RING
NPAD = 10240
RPT = NPAD // NS
NRB = RPT // C
DW = 16


def _pre_body(x_ref, z_ref, wt0_ref, bt0_ref, wt1_ref, bt1_ref,
              xm_ref, xself_ref):
    x = x_ref[...]
    x0 = jnp.maximum(jnp.dot(x, wt0_ref[...],
                             preferred_element_type=jnp.float32) + bt0_ref[...], 0.0)
    x1 = jnp.maximum(jnp.dot(x, wt1_ref[...],
                             preferred_element_type=jnp.float32) + bt1_ref[...], 0.0)
    x_out = ZR * x0 + (1.0 - ZR) * x1
    x_in = ZR * x1 + (1.0 - ZR) * x0
    mask = z_ref[...] > 0.5
    xm_ref[:N, :] = x_out[:, :DH]
    xm_ref[N:, :] = x_out[:, DH:]
    xself_ref[...] = jnp.where(mask, x_in, x_out)


def _post_body(accp_ref, degp_ref, xself_ref, z_ref,
               wc1a_ref, wc1b_ref, bc1_ref, wc0a_ref, wc0b_ref, bc0_ref,
               gnw_ref, gnb_ref, gnms_ref, out_ref):
    acc = jnp.concatenate([accp_ref[:N, :], accp_ref[NPAD:NPAD + N, :]], axis=1)
    deg = degp_ref[:N, :1]
    deg = jnp.where(deg < 0.5, deg + 1.0, deg)
    out = acc / deg
    mean = jnp.mean(out, axis=0, keepdims=True)
    o = out - mean * gnms_ref[...]
    var = jnp.mean(o * o, axis=0, keepdims=True)
    outn = gnw_ref[...] * o * lax.rsqrt(var + EPS) + gnb_ref[...]
    xself = xself_ref[...]
    y1 = (jnp.dot(outn, wc1a_ref[...], preferred_element_type=jnp.float32)
          + jnp.dot(xself, wc1b_ref[...], preferred_element_type=jnp.float32)
          + bc1_ref[...])
    y0 = (jnp.dot(outn, wc0a_ref[...], preferred_element_type=jnp.float32)
          + jnp.dot(xself, wc0b_ref[...], preferred_element_type=jnp.float32)
          + bc0_ref[...])
    y_in = ZR * y1 + (1.0 - ZR) * y0
    y_out = ZR * y0 + (1.0 - ZR) * y1
    mask = z_ref[...] > 0.5
    out_ref[...] = jnp.where(mask, y_in, y_out)


_sc_mesh = plsc.VectorSubcoreMesh(core_axis_name="c", subcore_axis_name="s")

_sc_params = pltpu.CompilerParams(use_tc_tiling_on_sc=False)
if "needs_layout_passes" in pltpu.CompilerParams.__dataclass_fields__:
    _sc_params = dataclasses.replace(_sc_params, needs_layout_passes=False)

_sc_scratch = (
    [pltpu.VMEM((C,), jnp.int32) for _ in range(RING)]
    + [pltpu.VMEM((C,), jnp.int32) for _ in range(RING)]
    + [pltpu.VMEM((C,), jnp.int32) for _ in range(RING)]
    + [pltpu.VMEM((C,), jnp.float32) for _ in range(RING)]
    + [pltpu.VMEM((C, DH), jnp.float32) for _ in range(RING)]
    + [pltpu.VMEM((C, DW), jnp.float32) for _ in range(RING)]
    + [pltpu.VMEM_SHARED((NPAD, DH), jnp.float32),
       pltpu.VMEM_SHARED((NPAD, DW), jnp.float32)]
    + [pltpu.SemaphoreType.DMA for _ in range(3 * RING)]
)


@functools.partial(
    pl.kernel,
    compiler_params=_sc_params,
    out_type=(jax.ShapeDtypeStruct((NC * NPAD, DH), jnp.float32),
              jax.ShapeDtypeStruct((NC * NPAD, DW), jnp.float32)),
    mesh=_sc_mesh,
    scratch_types=_sc_scratch,
)
def _sc_scatter(xm_hbm, row_hbm, col_hbm, w_hbm, zrow_hbm, zdeg_hbm,
                accp_hbm, degp_hbm, *scr):
    colvs = scr[0:RING]
    rowvs = scr[RING:2 * RING]
    rowscs = scr[2 * RING:3 * RING]
    wvs = scr[3 * RING:4 * RING]
    msgvs = scr[4 * RING:5 * RING]
    wrows = scr[5 * RING:6 * RING]
    acc_sh, deg_sh = scr[6 * RING], scr[6 * RING + 1]
    sem_i = scr[6 * RING + 2:6 * RING + 2 + RING]
    sem_g = scr[6 * RING + 2 + RING:6 * RING + 2 + 2 * RING]
    sem_s = scr[6 * RING + 2 + 2 * RING:6 * RING + 2 + 3 * RING]

    cid = lax.axis_index("c")
    sid = lax.axis_index("s")

    pltpu.sync_copy(zrow_hbm, msgvs[0])
    pltpu.sync_copy(zdeg_hbm, wrows[0])
    for rb in range(NRB):
        pltpu.sync_copy(msgvs[0], acc_sh.at[pl.ds(sid * RPT + rb * C, C)])
        pltpu.sync_copy(wrows[0], deg_sh.at[pl.ds(sid * RPT + rb * C, C)])
    plsc.subcore_barrier()

    base0 = sid * EPT
    half = cid * N

    def idx_start(b, ci):
        base = base0 + ci * C
        pltpu.async_copy(col_hbm.at[pl.ds(base, C)], colvs[b], sem_i[b])
        pltpu.async_copy(row_hbm.at[pl.ds(base, C)], rowvs[b], sem_i[b])
        pltpu.async_copy(w_hbm.at[pl.ds(base, C)], wvs[b], sem_i[b])

    def idx_wait(b, ci):
        base = base0 + ci * C
        pltpu.make_async_copy(col_hbm.at[pl.ds(base, C)], colvs[b], sem_i[b]).wait()
        pltpu.make_async_copy(row_hbm.at[pl.ds(base, C)], rowvs[b], sem_i[b]).wait()
        pltpu.make_async_copy(w_hbm.at[pl.ds(base, C)], wvs[b], sem_i[b]).wait()

    def shift_and_gather(b, ci):
        idx_wait(b, ci)

        @pl.loop(0, C, step=16)
        def _shift(k):
            sl = pl.ds(k, 16)
            colvs[b][sl] = colvs[b][sl] + half

        pltpu.async_copy(xm_hbm.at[colvs[b]], msgvs[b], sem_g[b])

    def gather_wait(b):
        pltpu.make_async_copy(xm_hbm.at[colvs[b]], msgvs[b], sem_g[b]).wait()

    def scale(b):
        @pl.loop(0, C, step=16)
        def _blk(e0):
            for j in range(16):
                w16 = plsc.load_gather(
                    wvs[b], [jnp.full((16,), j, jnp.int32) + e0])
                wrows[b][e0 + j, :] = w16
                for d in range(DH // 16):
                    sl = pl.ds(d * 16, 16)
                    msgvs[b][e0 + j, sl] = msgvs[b][e0 + j, sl] * w16

    def scatter_start(b):
        @pl.loop(0, C, step=16)
        def _snap(k):
            sl = pl.ds(k, 16)
            rowscs[b][sl] = rowvs[b][sl]

        pltpu.async_copy(msgvs[b], acc_sh.at[rowscs[b]], sem_s[b], add=True)
        pltpu.async_copy(wrows[b], deg_sh.at[rowscs[b]], sem_s[b], add=True)

    def scatter_wait(b):
        pltpu.make_async_copy(msgvs[b], acc_sh.at[rowscs[b]], sem_s[b]).wait()
        pltpu.make_async_copy(wrows[b], deg_sh.at[rowscs[b]], sem_s[b]).wait()

    def step(i, b, do_i3, do_sw, do_g2):
        b2 = (b + 2) % RING
        b3 = (b + 3) % RING
        if do_i3:
            idx_start(b3, i + 3)
        if do_sw:
            scatter_wait(b2)
        if do_g2:
            shift_and_gather(b2, i + 2)
        gather_wait(b)
        scale(b)
        scatter_start(b)

    idx_start(0, 0)
    idx_start(1, 1)
    idx_start(2, 2)
    shift_and_gather(0, 0)
    shift_and_gather(1, 1)

    for b in range(RING):
        step(b, b, True, b >= 3, True)

    @pl.loop(1, NGROUP - 1)
    def _group(g):
        for b in range(RING):
            step(g * RING + b, b, True, True, True)

    for b in range(RING):
        i = (NGROUP - 1) * RING + b
        step(i, b, i + 3 < NCHUNK, True, i + 2 < NCHUNK)

    for b in (2, 3, 4):
        scatter_wait(b)
    plsc.subcore_barrier()

    obase = cid * NPAD + sid * RPT
    for rb in range(NRB):
        m = msgvs[rb % RING]
        w = wrows[rb % RING]
        pltpu.sync_copy(acc_sh.at[pl.ds(sid * RPT + rb * C, C)], m)
        pltpu.sync_copy(m, accp_hbm.at[pl.ds(obase + rb * C, C)])
        pltpu.sync_copy(deg_sh.at[pl.ds(sid * RPT + rb * C, C)], w)
        pltpu.sync_copy(w, degp_hbm.at[pl.ds(obase + rb * C, C)])


def kernel(x_, edge_index, edge_weight, z, Wt0, bt0, Wt1, bt1,
           Wc0, bc0, Wc1, bc1, gn_w, gn_b, gn_ms):
    row = edge_index[0].astype(jnp.int32)
    col = edge_index[1].astype(jnp.int32)
    z2 = z.reshape(N, 1)

    xm2, xself = pl.pallas_call(
        _pre_body,
        out_shape=(jax.ShapeDtypeStruct((NC * N, DH), jnp.float32),
                   jax.ShapeDtypeStruct((N, D), jnp.float32)),
    )(x_, z2, Wt0, bt0.reshape(1, D), Wt1, bt1.reshape(1, D))

    zrow = jnp.zeros((C, DH), jnp.float32)
    zdeg = jnp.zeros((C, DW), jnp.float32)
    accp, degp = _sc_scatter(xm2, row, col, edge_weight, zrow, zdeg)

    out = pl.pallas_call(
        _post_body,
        out_shape=jax.ShapeDtypeStruct((N, D), jnp.float32),
    )(accp, degp, xself, z2,
      Wc1[:D], Wc1[D:], bc1.reshape(1, D),
      Wc0[:D], Wc0[D:], bc0.reshape(1, D),
      gn_w.reshape(1, D), gn_b.reshape(1, D), gn_ms.reshape(1, D))
    return out


# speedup vs baseline: 22.8097x; 2.2845x over previous
import dataclasses
import functools

import jax
import jax.numpy as jnp
from jax import lax
from jax.experimental import pallas as pl
from jax.experimental.pallas import tpu as pltpu
from jax.experimental.pallas import tpu_sc as plsc

N = 10000
E = 320000
D = 128
ZR = 0.8
EPS = 1e-5

NC = 2
NS = 16
DH = D // NC
EPT = E // NS
C = 80
NCHUNK = EPT // C
RING = 5
NGROUP = NCHUNK // RING
NPAD = 10240
RPT = NPAD // NS
NRB = RPT // C
DW = 16


def _pre_body(x_ref, z_ref, wt0_ref, bt0_ref, wt1_ref, bt1_ref,
              xm_ref, xself_ref):
    x = x_ref[...]
    x0 = jnp.maximum(jnp.dot(x, wt0_ref[...],
                             preferred_element_type=jnp.float32) + bt0_ref[...], 0.0)
    x1 = jnp.maximum(jnp.dot(x, wt1_ref[...],
                             preferred_element_type=jnp.float32) + bt1_ref[...], 0.0)
    x_out = ZR * x0 + (1.0 - ZR) * x1
    x_in = ZR * x1 + (1.0 - ZR) * x0
    mask = z_ref[...] > 0.5
    xm_ref[:N, :] = x_out[:, :DH]
    xm_ref[N:, :] = x_out[:, DH:]
    xself_ref[...] = jnp.where(mask, x_in, x_out)


def _post_body(accp_ref, degp_ref, xself_ref, z_ref,
               wc1a_ref, wc1b_ref, bc1_ref, wc0a_ref, wc0b_ref, bc0_ref,
               gnw_ref, gnb_ref, gnms_ref, out_ref):
    acc = jnp.concatenate([accp_ref[:N, :], accp_ref[NPAD:NPAD + N, :]], axis=1)
    deg = degp_ref[:N, :1]
    deg = jnp.where(deg < 0.5, deg + 1.0, deg)
    out = acc / deg
    mean = jnp.mean(out, axis=0, keepdims=True)
    o = out - mean * gnms_ref[...]
    var = jnp.mean(o * o, axis=0, keepdims=True)
    outn = gnw_ref[...] * o * lax.rsqrt(var + EPS) + gnb_ref[...]
    xself = xself_ref[...]
    y1 = (jnp.dot(outn, wc1a_ref[...], preferred_element_type=jnp.float32)
          + jnp.dot(xself, wc1b_ref[...], preferred_element_type=jnp.float32)
          + bc1_ref[...])
    y0 = (jnp.dot(outn, wc0a_ref[...], preferred_element_type=jnp.float32)
          + jnp.dot(xself, wc0b_ref[...], preferred_element_type=jnp.float32)
          + bc0_ref[...])
    y_in = ZR * y1 + (1.0 - ZR) * y0
    y_out = ZR * y0 + (1.0 - ZR) * y1
    mask = z_ref[...] > 0.5
    out_ref[...] = jnp.where(mask, y_in, y_out)


_sc_mesh = plsc.VectorSubcoreMesh(core_axis_name="c", subcore_axis_name="s")

_sc_params = pltpu.CompilerParams(use_tc_tiling_on_sc=False)
if "needs_layout_passes" in pltpu.CompilerParams.__dataclass_fields__:
    _sc_params = dataclasses.replace(_sc_params, needs_layout_passes=False)

_sc_scratch = (
    [pltpu.VMEM((C,), jnp.int32) for _ in range(RING)]
    + [pltpu.VMEM((C,), jnp.int32) for _ in range(RING)]
    + [pltpu.VMEM((C,), jnp.int32) for _ in range(RING)]
    + [pltpu.VMEM((C,), jnp.float32) for _ in range(RING)]
    + [pltpu.VMEM((C, DH), jnp.float32) for _ in range(RING)]
    + [pltpu.VMEM((C, DW), jnp.float32) for _ in range(RING)]
    + [pltpu.VMEM_SHARED((NPAD, DH), jnp.float32),
       pltpu.VMEM_SHARED((NPAD, DW), jnp.float32)]
    + [pltpu.SemaphoreType.DMA for _ in range(3 * RING)]
)


@functools.partial(
    pl.kernel,
    compiler_params=_sc_params,
    out_type=(jax.ShapeDtypeStruct((NC * NPAD, DH), jnp.float32),
              jax.ShapeDtypeStruct((NC * NPAD, DW), jnp.float32)),
    mesh=_sc_mesh,
    scratch_types=_sc_scratch,
)
def _sc_scatter(xm_hbm, row_hbm, col_hbm, w_hbm, zrow_hbm, zdeg_hbm,
                accp_hbm, degp_hbm, *scr):
    colvs = scr[0:RING]
    rowvs = scr[RING:2 * RING]
    rowscs = scr[2 * RING:3 * RING]
    wvs = scr[3 * RING:4 * RING]
    msgvs = scr[4 * RING:5 * RING]
    wrows = scr[5 * RING:6 * RING]
    acc_sh, deg_sh = scr[6 * RING], scr[6 * RING + 1]
    sem_i = scr[6 * RING + 2:6 * RING + 2 + RING]
    sem_g = scr[6 * RING + 2 + RING:6 * RING + 2 + 2 * RING]
    sem_s = scr[6 * RING + 2 + 2 * RING:6 * RING + 2 + 3 * RING]

    cid = lax.axis_index("c")
    sid = lax.axis_index("s")

    pltpu.sync_copy(zrow_hbm, msgvs[0])
    pltpu.sync_copy(zdeg_hbm, wrows[0])
    for rb in range(NRB):
        pltpu.sync_copy(msgvs[0], acc_sh.at[pl.ds(sid * RPT + rb * C, C)])
        pltpu.sync_copy(wrows[0], deg_sh.at[pl.ds(sid * RPT + rb * C, C)])
    plsc.subcore_barrier()

    base0 = sid * EPT
    half = cid * N

    def idx_start(b, ci):
        base = base0 + ci * C
        pltpu.async_copy(col_hbm.at[pl.ds(base, C)], colvs[b], sem_i[b])
        pltpu.async_copy(row_hbm.at[pl.ds(base, C)], rowvs[b], sem_i[b])
        pltpu.async_copy(w_hbm.at[pl.ds(base, C)], wvs[b], sem_i[b])

    def idx_wait(b, ci):
        base = base0 + ci * C
        pltpu.make_async_copy(col_hbm.at[pl.ds(base, C)], colvs[b], sem_i[b]).wait()
        pltpu.make_async_copy(row_hbm.at[pl.ds(base, C)], rowvs[b], sem_i[b]).wait()
        pltpu.make_async_copy(w_hbm.at[pl.ds(base, C)], wvs[b], sem_i[b]).wait()

    def shift_and_gather(b, ci):
        idx_wait(b, ci)

        @pl.loop(0, C, step=16)
        def _shift(k):
            sl = pl.ds(k, 16)
            colvs[b][sl] = colvs[b][sl] + half

        pltpu.async_copy(xm_hbm.at[colvs[b]], msgvs[b], sem_g[b])

    def gather_wait(b):
        pltpu.make_async_copy(xm_hbm.at[colvs[b]], msgvs[b], sem_g[b]).wait()

    def scale(b):
        @plsc.parallel_loop(0, C, 1, unroll=8)
        def _edge(e):
            w16 = plsc.load_gather(
                wvs[b], [jnp.full((16,), 0, jnp.int32) + e])
            wrows[b][e, :] = w16
            for d in range(DH // 16):
                sl = pl.ds(d * 16, 16)
                msgvs[b][e, sl] = msgvs[b][e, sl] * w16

    def scatter_start(b):
        @pl.loop(0, C, step=16)
        def _snap(k):
            sl = pl.ds(k, 16)
            rowscs[b][sl] = rowvs[b][sl]

        pltpu.async_copy(msgvs[b], acc_sh.at[rowscs[b]], sem_s[b], add=True)
        pltpu.async_copy(wrows[b], deg_sh.at[rowscs[b]], sem_s[b], add=True)

    def scatter_wait(b):
        pltpu.make_async_copy(msgvs[b], acc_sh.at[rowscs[b]], sem_s[b]).wait()
        pltpu.make_async_copy(wrows[b], deg_sh.at[rowscs[b]], sem_s[b]).wait()

    def step(i, b, do_i3, do_sw, do_g2):
        b2 = (b + 2) % RING
        b3 = (b + 3) % RING
        if do_i3:
            idx_start(b3, i + 3)
        if do_sw:
            scatter_wait(b2)
        if do_g2:
            shift_and_gather(b2, i + 2)
        gather_wait(b)
        scale(b)
        scatter_start(b)

    idx_start(0, 0)
    idx_start(1, 1)
    idx_start(2, 2)
    shift_and_gather(0, 0)
    shift_and_gather(1, 1)

    for b in range(RING):
        step(b, b, True, b >= 3, True)

    @pl.loop(1, NGROUP - 1)
    def _group(g):
        for b in range(RING):
            step(g * RING + b, b, True, True, True)

    for b in range(RING):
        i = (NGROUP - 1) * RING + b
        step(i, b, i + 3 < NCHUNK, True, i + 2 < NCHUNK)

    for b in (2, 3, 4):
        scatter_wait(b)
    plsc.subcore_barrier()

    obase = cid * NPAD + sid * RPT
    for rb in range(NRB):
        m = msgvs[rb % RING]
        w = wrows[rb % RING]
        pltpu.sync_copy(acc_sh.at[pl.ds(sid * RPT + rb * C, C)], m)
        pltpu.sync_copy(m, accp_hbm.at[pl.ds(obase + rb * C, C)])
        pltpu.sync_copy(deg_sh.at[pl.ds(sid * RPT + rb * C, C)], w)
        pltpu.sync_copy(w, degp_hbm.at[pl.ds(obase + rb * C, C)])


def kernel(x_, edge_index, edge_weight, z, Wt0, bt0, Wt1, bt1,
           Wc0, bc0, Wc1, bc1, gn_w, gn_b, gn_ms):
    row = edge_index[0].astype(jnp.int32)
    col = edge_index[1].astype(jnp.int32)
    z2 = z.reshape(N, 1)

    xm2, xself = pl.pallas_call(
        _pre_body,
        out_shape=(jax.ShapeDtypeStruct((NC * N, DH), jnp.float32),
                   jax.ShapeDtypeStruct((N, D), jnp.float32)),
    )(x_, z2, Wt0, bt0.reshape(1, D), Wt1, bt1.reshape(1, D))

    zrow = jnp.zeros((C, DH), jnp.float32)
    zdeg = jnp.zeros((C, DW), jnp.float32)
    accp, degp = _sc_scatter(xm2, row, col, edge_weight, zrow, zdeg)

    out = pl.pallas_call(
        _post_body,
        out_shape=jax.ShapeDtypeStruct((N, D), jnp.float32),
    )(accp, degp, xself, z2,
      Wc1[:D], Wc1[D:], bc1.reshape(1, D),
      Wc0[:D], Wc0[D:], bc0.reshape(1, D),
      gn_w.reshape(1, D), gn_b.reshape(1, D), gn_ms.reshape(1, D))
    return out
